# baseline (speedup 1.0000x reference)
import functools

import numpy as np
import jax
import jax.numpy as jnp
from jax import lax
from jax.experimental import pallas as pl
from jax.experimental.pallas import tpu as pltpu
from jax.experimental.pallas import tpu_sc as plsc

_VOCAB = 100000
_D = 128
_MAXLEN = 2048
_N_PARAM = 10000
_BATCH = 4
_SEQ = 2048
_EPS = 1e-12

_NC = 2
_NW = _NC * 16
_ROWS = _BATCH * _SEQ
_HROWS = _ROWS // 2
_HRPW = _HROWS // _NW
_TCB = 2048


def _make_pe_np():
    k = np.arange(_MAXLEN, dtype=np.float32)[:, None]
    div = np.exp(
        np.arange(0, _D, 2, dtype=np.float32) * (-np.log(_N_PARAM) / _D)
    )
    pe = np.zeros((_MAXLEN, _D), dtype=np.float32)
    pe[:, 0::2] = np.sin(k * div)
    pe[:, 1::2] = np.cos(k * div)
    return pe


_PE = _make_pe_np()


def _sc_gather_half_body(ids_hbm, table_hbm, out_hbm, idx_v, rows_v, gsem):
    c = lax.axis_index("c")
    s = lax.axis_index("s")
    wid = s * _NC + c
    base = wid * _HRPW
    b = base // _SEQ
    col = base % _SEQ

    pltpu.sync_copy(ids_hbm.at[b, pl.ds(col, _HRPW)], idx_v.at[0])
    pltpu.async_copy(table_hbm.at[idx_v.at[0]], rows_v, gsem).wait()
    pltpu.sync_copy(rows_v, out_hbm.at[pl.ds(base, _HRPW)])


def _tc_ln_body(x_ref, pe_ref, g_ref, b_ref, o_ref):
    x = x_ref[...] + pe_ref[...]
    m = jnp.mean(x, axis=-1, keepdims=True)
    v = jnp.mean(x * x, axis=-1, keepdims=True) - m * m
    y = (x - m) * lax.rsqrt(v + jnp.float32(_EPS))
    o_ref[...] = y * g_ref[...] + b_ref[...]


def _tc_ln_body_b(x_ref, pe_ref, g_ref, b_ref, prev_ref, o_ref):
    del prev_ref
    _tc_ln_body(x_ref, pe_ref, g_ref, b_ref, o_ref)


@jax.jit
def _embed_ln(ids, table, pe, gamma, beta):
    mesh = plsc.VectorSubcoreMesh(
        core_axis_name="c", subcore_axis_name="s", num_cores=_NC
    )
    sc_gather = pl.kernel(
        _sc_gather_half_body,
        out_type=jax.ShapeDtypeStruct((_HROWS, _D), jnp.float32),
        mesh=mesh,
        scratch_types=[
            pltpu.VMEM((1, _HRPW), jnp.int32),
            pltpu.VMEM((_HRPW, _D), jnp.float32),
            pltpu.SemaphoreType.DMA,
        ],
        compiler_params=pltpu.CompilerParams(needs_layout_passes=False),
    )
    g0 = sc_gather(ids[: _BATCH // 2], table)
    g1 = sc_gather(ids[_BATCH // 2 :], table)

    gview = gamma.reshape(1, _D)
    bview = beta.reshape(1, _D)
    n_half_blocks = _HROWS // _TCB

    ya = pl.pallas_call(
        _tc_ln_body,
        grid=(n_half_blocks,),
        in_specs=[
            pl.BlockSpec((_TCB, _D), lambda i: (i, 0)),
            pl.BlockSpec((_SEQ, _D), lambda i: (0, 0)),
            pl.BlockSpec((1, _D), lambda i: (0, 0)),
            pl.BlockSpec((1, _D), lambda i: (0, 0)),
        ],
        out_specs=pl.BlockSpec((_TCB, _D), lambda i: (i, 0)),
        out_shape=jax.ShapeDtypeStruct((_ROWS, _D), jnp.float32),
    )(g0, pe, gview, bview)

    return pl.pallas_call(
        _tc_ln_body_b,
        grid=(n_half_blocks,),
        in_specs=[
            pl.BlockSpec((_TCB, _D), lambda i: (i, 0)),
            pl.BlockSpec((_SEQ, _D), lambda i: (0, 0)),
            pl.BlockSpec((1, _D), lambda i: (0, 0)),
            pl.BlockSpec((1, _D), lambda i: (0, 0)),
            pl.BlockSpec(memory_space=pl.ANY),
        ],
        out_specs=pl.BlockSpec(
            (_TCB, _D), lambda i: (i + n_half_blocks, 0)
        ),
        out_shape=jax.ShapeDtypeStruct((_ROWS, _D), jnp.float32),
        input_output_aliases={4: 0},
    )(g1, pe, gview, bview, ya)


def kernel(input_ids, table, gamma, beta):
    pe = jnp.asarray(_PE)
    out = _embed_ln(input_ids, table, pe, gamma, beta)
    return out.reshape(_BATCH, _SEQ, _D)


# speedup vs baseline: 1.0666x; 1.0666x over previous
import functools

import numpy as np
import jax
import jax.numpy as jnp
from jax import lax
from jax.experimental import pallas as pl
from jax.experimental.pallas import tpu as pltpu
from jax.experimental.pallas import tpu_sc as plsc

_VOCAB = 100000
_D = 128
_MAXLEN = 2048
_N_PARAM = 10000
_BATCH = 4
_SEQ = 2048
_EPS = 1e-12

_NC = 2
_NW = _NC * 16
_ROWS = _BATCH * _SEQ
_RPW = _ROWS // _NW
_GCH = 128
_NCH = _RPW // _GCH
_TCB = 2048


def _make_pe_np():
    k = np.arange(_MAXLEN, dtype=np.float32)[:, None]
    div = np.exp(
        np.arange(0, _D, 2, dtype=np.float32) * (-np.log(_N_PARAM) / _D)
    )
    pe = np.zeros((_MAXLEN, _D), dtype=np.float32)
    pe[:, 0::2] = np.sin(k * div)
    pe[:, 1::2] = np.cos(k * div)
    return pe


_PE = _make_pe_np()


def _sc_gather_body(ids_hbm, table_hbm, out_hbm, idx_v, rows_v,
                    g0, g1, wsem):
    gsems = [g0, g1]
    c = lax.axis_index("c")
    s = lax.axis_index("s")
    wid = s * _NC + c
    base = wid * _RPW
    b = base // _SEQ
    col = base % _SEQ

    pltpu.sync_copy(ids_hbm.at[pl.ds(b, 1), pl.ds(col, _RPW)], idx_v)

    g_copies = [None] * _NCH
    w_copies = [None] * _NCH

    def _gather(j):
        return pltpu.async_copy(
            table_hbm.at[idx_v.at[0, pl.ds(j * _GCH, _GCH)]],
            rows_v.at[pl.ds(j * _GCH, _GCH)],
            gsems[j],
        )

    def _writeback(j):
        return pltpu.async_copy(
            rows_v.at[pl.ds(j * _GCH, _GCH)],
            out_hbm.at[pl.ds(base + j * _GCH, _GCH)],
            wsem,
        )

    g_copies[0] = _gather(0)
    g_copies[0].wait()
    g_copies[1] = _gather(1)
    w_copies[0] = _writeback(0)
    g_copies[1].wait()
    w_copies[1] = _writeback(1)
    w_copies[0].wait()
    w_copies[1].wait()


def _tc_ln_body(x_ref, pe_ref, g_ref, b_ref, o_ref):
    x = x_ref[...] + pe_ref[...]
    m = jnp.mean(x, axis=-1, keepdims=True)
    v = jnp.mean(x * x, axis=-1, keepdims=True) - m * m
    y = (x - m) * lax.rsqrt(v + jnp.float32(_EPS))
    o_ref[...] = y * g_ref[...] + b_ref[...]


@jax.jit
def _embed_ln(ids, table, pe, gamma, beta):
    mesh = plsc.VectorSubcoreMesh(
        core_axis_name="c", subcore_axis_name="s", num_cores=_NC
    )
    gathered = pl.kernel(
        _sc_gather_body,
        out_type=jax.ShapeDtypeStruct((_ROWS, _D), jnp.float32),
        mesh=mesh,
        scratch_types=[
            pltpu.VMEM((1, _RPW), jnp.int32),
            pltpu.VMEM((_RPW, _D), jnp.float32),
            pltpu.SemaphoreType.DMA,
            pltpu.SemaphoreType.DMA,
            pltpu.SemaphoreType.DMA,
        ],
        compiler_params=pltpu.CompilerParams(needs_layout_passes=False),
    )(ids, table)

    return pl.pallas_call(
        _tc_ln_body,
        grid=(_ROWS // _TCB,),
        in_specs=[
            pl.BlockSpec((_TCB, _D), lambda i: (i, 0)),
            pl.BlockSpec((_SEQ, _D), lambda i: (0, 0)),
            pl.BlockSpec((1, _D), lambda i: (0, 0)),
            pl.BlockSpec((1, _D), lambda i: (0, 0)),
        ],
        out_specs=pl.BlockSpec((_TCB, _D), lambda i: (i, 0)),
        out_shape=jax.ShapeDtypeStruct((_ROWS, _D), jnp.float32),
    )(gathered, pe, gamma.reshape(1, _D), beta.reshape(1, _D))


def kernel(input_ids, table, gamma, beta):
    pe = jnp.asarray(_PE)
    out = _embed_ln(input_ids, table, pe, gamma, beta)
    return out.reshape(_BATCH, _SEQ, _D)
